# baseline (speedup 1.0000x reference)
import functools

import jax
import jax.numpy as jnp
from jax import lax
from jax.experimental import pallas as pl
from jax.experimental.pallas import tpu as pltpu
from jax.experimental.pallas import tpu_sc as plsc

_S = 512
_SC = 128
_NJ = _S // _SC
_W = 640
_MAXW = 5


def _pool_kernel(lengths_ref, stepdiv_ref, feat_ref, feats_out_ref):
    b = pl.program_id(0)
    T = feat_ref.shape[-1]
    L = lengths_ref[b]
    sd = stepdiv_ref[0]
    off = T - L

    for j in range(_NJ):
        lo = (L * (j * _SC)) // sd + off
        t0 = jnp.minimum((lo // 128) * 128, T - _W)

        s = jax.lax.broadcasted_iota(jnp.int32, (1, _SC), 1) + j * _SC
        start_idx = (L * s) // sd + off
        end_idx = jnp.minimum((L * (s + 1) + sd - 1) // sd + off, T)
        counts = jnp.maximum(end_idx - start_idx, 1)

        t = jax.lax.broadcasted_iota(jnp.int32, (_W, _SC), 0) + t0
        in_win = (t - start_idx).astype(jnp.uint32) < counts.astype(jnp.uint32)
        sel = jnp.where(in_win, jnp.float32(1), jnp.float32(0))

        inv = 1.0 / counts.astype(jnp.float32)
        fwin = feat_ref[0, :, pl.ds(t0, _W)]
        feats_out_ref[0, :, j * _SC:(j + 1) * _SC] = (
            jnp.dot(fwin, sel, preferred_element_type=jnp.float32) * inv)


def _tc_pool(features, lengths, step_div):
    BN, C, T = features.shape
    grid_spec = pltpu.PrefetchScalarGridSpec(
        num_scalar_prefetch=2,
        grid=(BN,),
        in_specs=[pl.BlockSpec((1, C, T), lambda b, *_: (b, 0, 0))],
        out_specs=[pl.BlockSpec((1, C, _S), lambda b, *_: (b, 0, 0))],
    )
    return pl.pallas_call(
        _pool_kernel,
        grid_spec=grid_spec,
        compiler_params=pltpu.CompilerParams(
            dimension_semantics=("parallel",)),
        out_shape=[jax.ShapeDtypeStruct((BN, C, _S), features.dtype)],
    )(lengths, step_div, features)[0]


def _sc_mask_body(T, mask_hbm, lengths_hbm, stepdiv_hbm, out_hbm,
                  row_v, len_v, sd_v, out_v):
    wid = lax.axis_index("s") * 2 + lax.axis_index("c")
    b = wid // 2
    half = wid - 2 * (wid // 2)

    pltpu.sync_copy(mask_hbm.at[b], row_v)
    pltpu.sync_copy(lengths_hbm, len_v)
    pltpu.sync_copy(stepdiv_hbm, sd_v)

    bvec = jnp.zeros((16,), jnp.int32) + b
    L = plsc.load_gather(len_v, [bvec])
    sd = plsc.load_gather(sd_v, [bvec])
    off = T - L

    lanes = jax.lax.broadcasted_iota(jnp.int32, (16,), 0)
    for i in range(16):
        s = lanes + (half * 256 + i * 16)
        start = (L * s) // sd + off
        end = jnp.minimum((L * (s + 1) + sd - 1) // sd + off, T)
        counts = jnp.maximum(end - start, 1)
        acc = jnp.zeros((16,), jnp.float32)
        for w in range(_MAXW):
            idx = jnp.minimum(start + w, T - 1)
            g = plsc.load_gather(row_v, [idx])
            acc = acc + jnp.where(w < counts, g, 0.0)
        out_v[pl.ds(i * 16, 16)] = acc / counts.astype(jnp.float32)

    pltpu.sync_copy(out_v, out_hbm.at[b, pl.ds(half * 256, 256)])


def _sc_mask_pool(mask2d, lengths, step_div):
    BN, T = mask2d.shape
    mesh = plsc.VectorSubcoreMesh(core_axis_name="c", subcore_axis_name="s")
    kern = functools.partial(
        pl.kernel,
        mesh=mesh,
        compiler_params=pltpu.CompilerParams(needs_layout_passes=False),
        out_type=jax.ShapeDtypeStruct((BN, _S), jnp.float32),
        scratch_types=[
            pltpu.VMEM((T,), jnp.float32),
            pltpu.VMEM((BN,), jnp.int32),
            pltpu.VMEM((16,), jnp.int32),
            pltpu.VMEM((256,), jnp.float32),
        ],
    )(functools.partial(_sc_mask_body, T))
    sd_vec = jnp.broadcast_to(step_div, (16,)).astype(jnp.int32)
    return kern(mask2d, lengths, sd_vec)


def kernel(features, mask, valid_lengths, target_len):
    BN, C, T = features.shape
    lengths = jnp.clip(valid_lengths.astype(jnp.int32), 1, T)
    step_div = jnp.maximum(jnp.asarray(target_len, jnp.int32), 1).reshape(1)

    pooled_mask = _sc_mask_pool(mask.reshape(BN, T), lengths, step_div)
    pooled_feats = _tc_pool(features, lengths, step_div)
    return pooled_feats, pooled_mask.reshape(BN, 1, _S).astype(mask.dtype)


# speedup vs baseline: 1.9694x; 1.9694x over previous
import jax
import jax.numpy as jnp
from jax.experimental import pallas as pl
from jax.experimental.pallas import tpu as pltpu

_S = 512
_SC = 128
_NJ = _S // _SC
_BB = 2
_W = 640


def _pool_kernel(lengths_ref, stepdiv_ref, feat_ref, mask_ref,
                 feats_out_ref, mask_out_ref):
    g = pl.program_id(0)
    T = feat_ref.shape[-1]
    sd = stepdiv_ref[0]
    for bb in range(_BB):
      b = g * _BB + bb
      L = lengths_ref[b]
      off = T - L
      for j in range(_NJ):
        lo = (L * (j * _SC)) // sd + off
        t0 = jnp.minimum((lo // 128) * 128, T - _W)

        s = jax.lax.broadcasted_iota(jnp.int32, (1, _SC), 1) + j * _SC
        start_idx = (L * s) // sd + off
        end_idx = jnp.minimum((L * (s + 1) + sd - 1) // sd + off, T)
        counts = jnp.maximum(end_idx - start_idx, 1)

        t = jax.lax.broadcasted_iota(jnp.int32, (_W, _SC), 0) + t0
        in_win = (t - start_idx).astype(jnp.uint32) < counts.astype(jnp.uint32)
        sel = jnp.where(in_win, jnp.float32(1), jnp.float32(0))

        inv = 1.0 / counts.astype(jnp.float32)
        fwin = feat_ref[bb, :, pl.ds(t0, _W)]
        feats_out_ref[bb, :, j * _SC:(j + 1) * _SC] = (
            jnp.dot(fwin, sel, preferred_element_type=jnp.float32) * inv)
        mwin = mask_ref[bb, :, pl.ds(t0, _W)]
        mask_out_ref[bb, :, j * _SC:(j + 1) * _SC] = (
            jnp.dot(mwin, sel, preferred_element_type=jnp.float32) * inv)


def kernel(features, mask, valid_lengths, target_len):
    BN, C, T = features.shape
    lengths = jnp.clip(valid_lengths.astype(jnp.int32), 1, T)
    step_div = jnp.maximum(jnp.asarray(target_len, jnp.int32), 1).reshape(1)

    grid_spec = pltpu.PrefetchScalarGridSpec(
        num_scalar_prefetch=2,
        grid=(BN // _BB,),
        in_specs=[
            pl.BlockSpec((_BB, C, T), lambda b, *_: (b, 0, 0)),
            pl.BlockSpec((_BB, 1, T), lambda b, *_: (b, 0, 0)),
        ],
        out_specs=[
            pl.BlockSpec((_BB, C, _S), lambda b, *_: (b, 0, 0)),
            pl.BlockSpec((_BB, 1, _S), lambda b, *_: (b, 0, 0)),
        ],
    )
    pooled_feats, pooled_mask = pl.pallas_call(
        _pool_kernel,
        grid_spec=grid_spec,
        compiler_params=pltpu.CompilerParams(
            dimension_semantics=("parallel",)),
        out_shape=[
            jax.ShapeDtypeStruct((BN, C, _S), features.dtype),
            jax.ShapeDtypeStruct((BN, 1, _S), mask.dtype),
        ],
    )(lengths, step_div, features, mask)
    return pooled_feats, pooled_mask


# speedup vs baseline: 2.0441x; 1.0379x over previous
import jax
import jax.numpy as jnp
from jax.experimental import pallas as pl
from jax.experimental.pallas import tpu as pltpu

_S = 512
_SC = 128
_NJ = _S // _SC
_BB = 4
_W = 640


def _pool_kernel(lengths_ref, stepdiv_ref, feat_ref, mask_ref,
                 feats_out_ref, mask_out_ref):
    g = pl.program_id(0)
    T = feat_ref.shape[-1]
    sd = stepdiv_ref[0]
    for bb in range(_BB):
      b = g * _BB + bb
      L = lengths_ref[b]
      off = T - L
      for j in range(_NJ):
        lo = (L * (j * _SC)) // sd + off
        t0 = jnp.minimum((lo // 128) * 128, T - _W)

        s = jax.lax.broadcasted_iota(jnp.int32, (1, _SC), 1) + j * _SC
        start_idx = (L * s) // sd + off
        end_idx = jnp.minimum((L * (s + 1) + sd - 1) // sd + off, T)
        counts = jnp.maximum(end_idx - start_idx, 1)

        t = jax.lax.broadcasted_iota(jnp.int32, (_W, _SC), 0) + t0
        in_win = (t - start_idx).astype(jnp.uint32) < counts.astype(jnp.uint32)
        sel = jnp.where(in_win, jnp.float32(1), jnp.float32(0))

        inv = 1.0 / counts.astype(jnp.float32)
        fwin = feat_ref[bb, :, pl.ds(t0, _W)]
        feats_out_ref[bb, :, j * _SC:(j + 1) * _SC] = (
            jnp.dot(fwin, sel, preferred_element_type=jnp.float32) * inv)
        mwin = mask_ref[bb, :, pl.ds(t0, _W)]
        mask_out_ref[bb, :, j * _SC:(j + 1) * _SC] = (
            jnp.dot(mwin, sel, preferred_element_type=jnp.float32) * inv)


def kernel(features, mask, valid_lengths, target_len):
    BN, C, T = features.shape
    lengths = jnp.clip(valid_lengths.astype(jnp.int32), 1, T)
    step_div = jnp.maximum(jnp.asarray(target_len, jnp.int32), 1).reshape(1)

    grid_spec = pltpu.PrefetchScalarGridSpec(
        num_scalar_prefetch=2,
        grid=(BN // _BB,),
        in_specs=[
            pl.BlockSpec((_BB, C, T), lambda b, *_: (b, 0, 0)),
            pl.BlockSpec((_BB, 1, T), lambda b, *_: (b, 0, 0)),
        ],
        out_specs=[
            pl.BlockSpec((_BB, C, _S), lambda b, *_: (b, 0, 0)),
            pl.BlockSpec((_BB, 1, _S), lambda b, *_: (b, 0, 0)),
        ],
    )
    pooled_feats, pooled_mask = pl.pallas_call(
        _pool_kernel,
        grid_spec=grid_spec,
        compiler_params=pltpu.CompilerParams(
            dimension_semantics=("parallel",)),
        out_shape=[
            jax.ShapeDtypeStruct((BN, C, _S), features.dtype),
            jax.ShapeDtypeStruct((BN, 1, _S), mask.dtype),
        ],
    )(lengths, step_div, features, mask)
    return pooled_feats, pooled_mask
